# baseline (speedup 1.0000x reference)
import functools

import jax
import jax.numpy as jnp
from jax import lax
from jax.experimental import pallas as pl
from jax.experimental.pallas import tpu as pltpu
from jax.experimental.pallas import tpu_sc as plsc

_EPS = 1e-5

_NUM_SC = 2
_NUM_SUBCORES = 16
_LANES = 16


def _linearize_emb(emb, vp2, eb):
    V, D = emb.shape

    def body(in_ref, out_ref):
        x = in_ref[...]
        out_ref[...] = jnp.concatenate([x[:eb], x[eb:]], axis=1)

    pairs = pl.pallas_call(
        body,
        grid=(vp2 // eb,),
        in_specs=[pl.BlockSpec((2 * eb, D), lambda i: (i, 0))],
        out_specs=pl.BlockSpec((eb, 2 * D), lambda i: (i, 0)),
        out_shape=jax.ShapeDtypeStruct((vp2, 2 * D), jnp.float32),
    )(emb)
    return pairs.reshape(vp2 * 2, D)


def _pad_idx(idx, hp, eb):
    B, H = idx.shape
    IB = 512
    lo = eb - 1
    hi = ~(2 * eb - 1)
    sh = eb.bit_length() - 1

    def body(in_ref, out_ref):
        x = in_ref[...]
        x = (x & hi) + ((x & lo) << 1) + ((x >> sh) & 1)
        out_ref[...] = jnp.concatenate(
            [x, jnp.zeros((IB, hp - H), jnp.int32)], axis=1)

    return pl.pallas_call(
        body,
        grid=(B // IB,),
        in_specs=[pl.BlockSpec((IB, H), lambda i: (i, 0))],
        out_specs=pl.BlockSpec((IB, hp), lambda i: (i, 0)),
        out_shape=jax.ShapeDtypeStruct((B, hp), jnp.int32),
    )(idx)


@functools.lru_cache(maxsize=None)
def _make_pool(B, H, HP, VP, D):
    NW = _NUM_SC * _NUM_SUBCORES
    BPW = B // NW
    assert B % NW == 0 and BPW % 2 == 0 and D % _LANES == 0
    C0 = min(104, H)
    C1 = H - C0
    NCH = D // _LANES
    mesh = plsc.VectorSubcoreMesh(core_axis_name="c", subcore_axis_name="s")

    @functools.partial(
        pl.kernel,
        out_type=jax.ShapeDtypeStruct((B, D), jnp.float32),
        mesh=mesh,
        scratch_types=[
            pltpu.VMEM((BPW, HP), jnp.int32),
            pltpu.VMEM((4, H, D), jnp.float32),
            pltpu.VMEM((BPW, D), jnp.float32),
            pltpu.SemaphoreType.DMA,
            pltpu.SemaphoreType.DMA,
            pltpu.SemaphoreType.DMA,
            pltpu.SemaphoreType.DMA,
            pltpu.SemaphoreType.DMA,
        ],
        compiler_params=pltpu.CompilerParams(use_tc_tiling_on_sc=False),
    )
    def pool(idx_hbm, emb_hbm, out_hbm, idx_v, rows_v, out_v, sem_i,
             sem0, sem1, sem2, sem3):
        wid = lax.axis_index("s") * _NUM_SC + lax.axis_index("c")
        base = wid * BPW
        pltpu.async_copy(idx_hbm.at[pl.ds(base, BPW)], idx_v, sem_i).wait()
        sems = (sem0, sem1, sem2, sem3)

        def fire(b, p):
            pltpu.make_async_copy(
                emb_hbm.at[idx_v.at[b, pl.ds(0, C0)]],
                rows_v.at[p, pl.ds(0, C0)], sems[p]).start()
            if C1:
                pltpu.make_async_copy(
                    emb_hbm.at[idx_v.at[b, pl.ds(C0, C1)]],
                    rows_v.at[p, pl.ds(C0, C1)], sems[p]).start()

        def wait(p):
            pltpu.make_async_copy(
                emb_hbm.at[idx_v.at[0, pl.ds(0, C0)]],
                rows_v.at[p, pl.ds(0, C0)], sems[p]).wait()
            if C1:
                pltpu.make_async_copy(
                    emb_hbm.at[idx_v.at[0, pl.ds(C0, C1)]],
                    rows_v.at[p, pl.ds(C0, C1)], sems[p]).wait()

        def reduce(b, p):
            def body(i, accs):
                return tuple(
                    a + rows_v[p, i, pl.ds(_LANES * c, _LANES)]
                    for c, a in enumerate(accs)
                )
            init = tuple(jnp.zeros((_LANES,), jnp.float32) for _ in range(NCH))
            accs = lax.fori_loop(0, H, body, init, unroll=8)
            for c in range(NCH):
                out_v[b, pl.ds(_LANES * c, _LANES)] = accs[c]

        for p in range(4):
            fire(p, p)

        @pl.loop(0, BPW, step=4)
        def _(b):
            for p in range(4):
                wait(p)
                reduce(b + p, p)

                @pl.when(b + p + 4 < BPW)
                def _():
                    fire(b + p + 4, p)

        pltpu.sync_copy(out_v, out_hbm.at[pl.ds(base, BPW)])

    return pool


def _mlp_body(idx_ref, pooled_ref, w1_ref, w1r_ref, c1_ref, w2_ref, c2_ref,
              loc_ref, scale_ref):
    D = pooled_ref.shape[1]
    cnt = jnp.sum((idx_ref[...] > 0).astype(jnp.float32), axis=1, keepdims=True)
    x = pooled_ref[...] / cnt
    h = jnp.dot(x, w1_ref[...], preferred_element_type=jnp.float32)
    h = h + jnp.log(cnt) * w1r_ref[...] + c1_ref[...]
    h = jnp.maximum(h, 0.0)
    o = jnp.dot(h, w2_ref[...], preferred_element_type=jnp.float32) + c2_ref[...]
    loc_ref[...] = o[:, :D]
    scale_ref[...] = jax.nn.softplus(o[:, D:])


def kernel(idx, read_depth, emb, W1, b1, bn1_g, bn1_b, bn1_m, bn1_v,
           W2, b2, bn2_g, bn2_b, bn2_m, bn2_v):
    del read_depth
    idx = idx.astype(jnp.int32)
    B, H = idx.shape
    D = emb.shape[1]
    HID = W1.shape[1]

    a1 = bn1_g / jnp.sqrt(bn1_v + _EPS)
    W1s = W1 * a1[None, :]
    c1 = ((b1 - bn1_m) * a1 + bn1_b)[None, :]
    a2 = bn2_g / jnp.sqrt(bn2_v + _EPS)
    W2s = W2 * a2[None, :]
    c2 = ((b2 - bn2_m) * a2 + bn2_b)[None, :]

    HP = 256
    VP2 = 50176
    embL = _linearize_emb(emb, VP2, 1024)
    idxP = _pad_idx(idx, HP, 1024)
    pooled = _make_pool(B, H, HP, VP2 * 2, D)(idxP, embL)

    BB = 512
    grid = (B // BB,)
    loc, scale = pl.pallas_call(
        _mlp_body,
        grid=grid,
        in_specs=[
            pl.BlockSpec((BB, H), lambda i: (i, 0)),
            pl.BlockSpec((BB, D), lambda i: (i, 0)),
            pl.BlockSpec((D, HID), lambda i: (0, 0)),
            pl.BlockSpec((1, HID), lambda i: (0, 0)),
            pl.BlockSpec((1, HID), lambda i: (0, 0)),
            pl.BlockSpec((HID, 2 * D), lambda i: (0, 0)),
            pl.BlockSpec((1, 2 * D), lambda i: (0, 0)),
        ],
        out_specs=[
            pl.BlockSpec((BB, D), lambda i: (i, 0)),
            pl.BlockSpec((BB, D), lambda i: (i, 0)),
        ],
        out_shape=[
            jax.ShapeDtypeStruct((B, D), jnp.float32),
            jax.ShapeDtypeStruct((B, D), jnp.float32),
        ],
    )(idx, pooled, W1s[:D], W1s[D:D + 1], c1, W2s, c2)
    return loc, scale


# speedup vs baseline: 1.0127x; 1.0127x over previous
import functools

import jax
import jax.numpy as jnp
from jax import lax
from jax.experimental import pallas as pl
from jax.experimental.pallas import tpu as pltpu
from jax.experimental.pallas import tpu_sc as plsc

_EPS = 1e-5

_NUM_SC = 2
_NUM_SUBCORES = 16
_LANES = 16
_NW = _NUM_SC * _NUM_SUBCORES


@functools.lru_cache(maxsize=None)
def _make_repack(V, D):
    BR = 64
    NFULL = V // BR
    REM = V - NFULL * BR
    REM_PAD = -(-REM // 8) * 8 if REM else 0
    VP = NFULL * BR + REM_PAD
    SLOTS = -(-NFULL // _NW)
    LAST_W = NFULL % _NW
    mesh = plsc.VectorSubcoreMesh(core_axis_name="c", subcore_axis_name="s")

    @functools.partial(
        pl.kernel,
        out_type=jax.ShapeDtypeStruct((VP * D,), jnp.float32),
        mesh=mesh,
        scratch_types=[
            pltpu.VMEM((2, BR, D), jnp.float32),
            pltpu.VMEM((2, BR * D), jnp.float32),
            pltpu.SemaphoreType.DMA,
            pltpu.SemaphoreType.DMA,
            pltpu.SemaphoreType.DMA,
            pltpu.SemaphoreType.DMA,
        ],
        compiler_params=pltpu.CompilerParams(disable_bounds_checks=True),
    )
    def repack(emb_hbm, out_hbm, bin_, bout, si0, si1, so0, so1):
        wid = lax.axis_index("s") * _NUM_SC + lax.axis_index("c")
        sin = (si0, si1)
        sout = (so0, so1)

        def load(k, p, rows):
            pltpu.make_async_copy(
                emb_hbm.at[pl.ds(k * BR, rows)],
                bin_.at[p, pl.ds(0, rows)],
                sin[p]).start()

        def wait_in(p, rows):
            pltpu.make_async_copy(
                emb_hbm.at[pl.ds(0, rows)],
                bin_.at[p, pl.ds(0, rows)],
                sin[p]).wait()

        def tec_copy(p, rows):
            @pl.loop(0, rows)
            def _(r):
                for c in range(D // _LANES):
                    bout[p, pl.ds(r * D + _LANES * c, _LANES)] = (
                        bin_[p, r, pl.ds(_LANES * c, _LANES)])

        def store(k, p, rows):
            pltpu.make_async_copy(
                bout.at[p, pl.ds(0, rows * D)],
                out_hbm.at[pl.ds(k * BR * D, rows * D)],
                sout[p]).start()

        def wait_out(p, rows):
            pltpu.make_async_copy(
                bout.at[p, pl.ds(0, rows * D)],
                out_hbm.at[pl.ds(0, rows * D)],
                sout[p]).wait()

        @pl.when(wid < NFULL)
        def _():
            load(wid, 0, BR)

        @pl.loop(0, (SLOTS + 1) // 2)
        def _(jj):
            for p in range(2):
                j = 2 * jj + p
                k = wid + _NW * j

                @pl.when(k < NFULL)
                def _():
                    wait_in(p, BR)
                    kn = wid + _NW * (j + 1)

                    @pl.when(kn < NFULL)
                    def _():
                        load(kn, 1 - p, BR)

                    @pl.when(j >= 2)
                    def _():
                        wait_out(p, BR)

                    tec_copy(p, BR)
                    store(k, p, BR)

        for p in range(2):
            @pl.when(wid + _NW * p < NFULL)
            def _():
                wait_out(p, BR)

        if REM:
            @pl.when(wid == LAST_W)
            def _():
                start = NFULL * BR + 0 * wid
                pltpu.make_async_copy(
                    emb_hbm.at[pl.ds(start, REM_PAD)],
                    bin_.at[0, pl.ds(0, REM_PAD)],
                    sin[0]).start()
                wait_in(0, REM_PAD)
                tec_copy(0, REM_PAD)
                store(NFULL, 0, REM_PAD)
                wait_out(0, REM_PAD)

    return repack, VP


@functools.lru_cache(maxsize=None)
def _make_pool(B, H, VP, D):
    BPW = B // _NW
    assert B % _NW == 0 and BPW % 4 == 0 and D % _LANES == 0
    C0 = min(104, H)
    C1 = H - C0
    NCH = D // _LANES
    mesh = plsc.VectorSubcoreMesh(core_axis_name="c", subcore_axis_name="s")

    @functools.partial(
        pl.kernel,
        out_type=jax.ShapeDtypeStruct((B, D), jnp.float32),
        mesh=mesh,
        scratch_types=[
            pltpu.VMEM((BPW, H), jnp.int32),
            pltpu.VMEM((4, H, D), jnp.float32),
            pltpu.VMEM((BPW, D), jnp.float32),
            pltpu.SemaphoreType.DMA,
            pltpu.SemaphoreType.DMA,
            pltpu.SemaphoreType.DMA,
            pltpu.SemaphoreType.DMA,
            pltpu.SemaphoreType.DMA,
        ],
        compiler_params=pltpu.CompilerParams(use_tc_tiling_on_sc=False),
    )
    def pool(idx_hbm, emb_hbm, out_hbm, idx_v, rows_v, out_v, sem_i,
             sem0, sem1, sem2, sem3):
        wid = lax.axis_index("s") * _NUM_SC + lax.axis_index("c")
        base = wid * BPW
        pltpu.async_copy(idx_hbm.at[pl.ds(base, BPW)], idx_v, sem_i).wait()
        sems = (sem0, sem1, sem2, sem3)

        def fire(b, p):
            pltpu.make_async_copy(
                emb_hbm.at[idx_v.at[b, pl.ds(0, C0)]],
                rows_v.at[p, pl.ds(0, C0)], sems[p]).start()
            if C1:
                pltpu.make_async_copy(
                    emb_hbm.at[idx_v.at[b, pl.ds(C0, C1)]],
                    rows_v.at[p, pl.ds(C0, C1)], sems[p]).start()

        def wait(p):
            pltpu.make_async_copy(
                emb_hbm.at[idx_v.at[0, pl.ds(0, C0)]],
                rows_v.at[p, pl.ds(0, C0)], sems[p]).wait()
            if C1:
                pltpu.make_async_copy(
                    emb_hbm.at[idx_v.at[0, pl.ds(C0, C1)]],
                    rows_v.at[p, pl.ds(C0, C1)], sems[p]).wait()

        def reduce(b, p):
            def body(i, accs):
                return tuple(
                    a + rows_v[p, i, pl.ds(_LANES * c, _LANES)]
                    for c, a in enumerate(accs)
                )
            init = tuple(jnp.zeros((_LANES,), jnp.float32) for _ in range(NCH))
            accs = lax.fori_loop(0, H, body, init, unroll=8)
            for c in range(NCH):
                out_v[b, pl.ds(_LANES * c, _LANES)] = accs[c]

        for p in range(4):
            fire(p, p)

        @pl.loop(0, BPW, step=4)
        def _(b):
            for p in range(4):
                wait(p)
                reduce(b + p, p)

                @pl.when(b + p + 4 < BPW)
                def _():
                    fire(b + p + 4, p)

        pltpu.sync_copy(out_v, out_hbm.at[pl.ds(base, BPW)])

    return pool


def _mlp_body(idx_ref, pooled_ref, w1_ref, w1r_ref, c1_ref, w2_ref, c2_ref,
              loc_ref, scale_ref):
    D = pooled_ref.shape[1]
    cnt = jnp.sum((idx_ref[...] > 0).astype(jnp.float32), axis=1, keepdims=True)
    x = pooled_ref[...] / cnt
    h = jnp.dot(x, w1_ref[...], preferred_element_type=jnp.float32)
    h = h + jnp.log(cnt) * w1r_ref[...] + c1_ref[...]
    h = jnp.maximum(h, 0.0)
    o = jnp.dot(h, w2_ref[...], preferred_element_type=jnp.float32) + c2_ref[...]
    loc_ref[...] = o[:, :D]
    scale_ref[...] = jax.nn.softplus(o[:, D:])


def kernel(idx, read_depth, emb, W1, b1, bn1_g, bn1_b, bn1_m, bn1_v,
           W2, b2, bn2_g, bn2_b, bn2_m, bn2_v):
    del read_depth
    idx = idx.astype(jnp.int32)
    B, H = idx.shape
    V, D = emb.shape
    HID = W1.shape[1]

    a1 = bn1_g / jnp.sqrt(bn1_v + _EPS)
    W1s = W1 * a1[None, :]
    c1 = ((b1 - bn1_m) * a1 + bn1_b)[None, :]
    a2 = bn2_g / jnp.sqrt(bn2_v + _EPS)
    W2s = W2 * a2[None, :]
    c2 = ((b2 - bn2_m) * a2 + bn2_b)[None, :]

    repack, VP = _make_repack(V, D)
    emb_lin = repack(emb).reshape(VP, D)
    pooled = _make_pool(B, H, VP, D)(idx, emb_lin)

    BB = 512
    grid = (B // BB,)
    loc, scale = pl.pallas_call(
        _mlp_body,
        grid=grid,
        in_specs=[
            pl.BlockSpec((BB, H), lambda i: (i, 0)),
            pl.BlockSpec((BB, D), lambda i: (i, 0)),
            pl.BlockSpec((D, HID), lambda i: (0, 0)),
            pl.BlockSpec((1, HID), lambda i: (0, 0)),
            pl.BlockSpec((1, HID), lambda i: (0, 0)),
            pl.BlockSpec((HID, 2 * D), lambda i: (0, 0)),
            pl.BlockSpec((1, 2 * D), lambda i: (0, 0)),
        ],
        out_specs=[
            pl.BlockSpec((BB, D), lambda i: (i, 0)),
            pl.BlockSpec((BB, D), lambda i: (i, 0)),
        ],
        out_shape=[
            jax.ShapeDtypeStruct((B, D), jnp.float32),
            jax.ShapeDtypeStruct((B, D), jnp.float32),
        ],
    )(idx, pooled, W1s[:D], W1s[D:D + 1], c1, W2s, c2)
    return loc, scale


# speedup vs baseline: 1.2147x; 1.1995x over previous
import functools

import jax
import jax.numpy as jnp
from jax import lax
from jax.experimental import pallas as pl
from jax.experimental.pallas import tpu as pltpu
from jax.experimental.pallas import tpu_sc as plsc

_EPS = 1e-5

_NUM_SC = 2
_NUM_SUBCORES = 16
_LANES = 16
_NW = _NUM_SC * _NUM_SUBCORES
_NBUF = 4


@functools.lru_cache(maxsize=None)
def _make_pool(B, H, D):
    BPW = B // _NW
    assert B % _NW == 0 and BPW % _NBUF == 0 and D % _LANES == 0
    C0 = min(104, H)
    C1 = H - C0
    NCH = D // _LANES
    mesh = plsc.VectorSubcoreMesh(core_axis_name="c", subcore_axis_name="s")

    @functools.partial(
        pl.kernel,
        out_type=jax.ShapeDtypeStruct((B, D), jnp.float32),
        mesh=mesh,
        scratch_types=[
            pltpu.VMEM((BPW, H), jnp.int32),
            pltpu.VMEM((_NBUF, H, D), jnp.float32),
            pltpu.VMEM((BPW, D), jnp.float32),
            pltpu.SemaphoreType.DMA,
        ] + [pltpu.SemaphoreType.DMA] * _NBUF,
        compiler_params=pltpu.CompilerParams(use_tc_tiling_on_sc=False),
    )
    def pool(idx_hbm, emb_hbm, out_hbm, idx_v, rows_v, out_v, sem_i, *sems):
        wid = lax.axis_index("s") * _NUM_SC + lax.axis_index("c")
        base = wid * BPW
        pltpu.async_copy(idx_hbm.at[pl.ds(base, BPW)], idx_v, sem_i).wait()

        def fire(b, p):
            pltpu.make_async_copy(
                emb_hbm.at[idx_v.at[b, pl.ds(0, C0)]],
                rows_v.at[p, pl.ds(0, C0)], sems[p]).start()
            if C1:
                pltpu.make_async_copy(
                    emb_hbm.at[idx_v.at[b, pl.ds(C0, C1)]],
                    rows_v.at[p, pl.ds(C0, C1)], sems[p]).start()

        def wait(p):
            pltpu.make_async_copy(
                emb_hbm.at[idx_v.at[0, pl.ds(0, C0)]],
                rows_v.at[p, pl.ds(0, C0)], sems[p]).wait()
            if C1:
                pltpu.make_async_copy(
                    emb_hbm.at[idx_v.at[0, pl.ds(C0, C1)]],
                    rows_v.at[p, pl.ds(C0, C1)], sems[p]).wait()

        def reduce(b, p):
            def body(i, accs):
                return tuple(
                    a + rows_v[p, i, pl.ds(_LANES * c, _LANES)]
                    for c, a in enumerate(accs)
                )
            init = tuple(jnp.zeros((_LANES,), jnp.float32) for _ in range(NCH))
            accs = lax.fori_loop(0, H, body, init, unroll=8)
            for c in range(NCH):
                out_v[b, pl.ds(_LANES * c, _LANES)] = accs[c]

        for p in range(_NBUF):
            fire(p, p)

        @pl.loop(0, BPW, step=_NBUF)
        def _(b):
            for p in range(_NBUF):
                wait(p)
                reduce(b + p, p)

                @pl.when(b + p + _NBUF < BPW)
                def _():
                    fire(b + p + _NBUF, p)

        pltpu.sync_copy(out_v, out_hbm.at[pl.ds(base, BPW)])

    return pool


def _mlp_body(idx_ref, pooled_ref, w1_ref, w1r_ref, c1_ref, w2_ref, c2_ref,
              loc_ref, scale_ref):
    D = pooled_ref.shape[1]
    cnt = jnp.sum((idx_ref[...] > 0).astype(jnp.float32), axis=1, keepdims=True)
    x = pooled_ref[...] / cnt
    h = jnp.dot(x, w1_ref[...], preferred_element_type=jnp.float32)
    h = h + jnp.log(cnt) * w1r_ref[...] + c1_ref[...]
    h = jnp.maximum(h, 0.0)
    o = jnp.dot(h, w2_ref[...], preferred_element_type=jnp.float32) + c2_ref[...]
    loc_ref[...] = o[:, :D]
    scale_ref[...] = jax.nn.softplus(o[:, D:])


def kernel(idx, read_depth, emb, W1, b1, bn1_g, bn1_b, bn1_m, bn1_v,
           W2, b2, bn2_g, bn2_b, bn2_m, bn2_v):
    del read_depth
    idx = idx.astype(jnp.int32)
    B, H = idx.shape
    D = emb.shape[1]
    HID = W1.shape[1]

    a1 = bn1_g / jnp.sqrt(bn1_v + _EPS)
    W1s = W1 * a1[None, :]
    c1 = ((b1 - bn1_m) * a1 + bn1_b)[None, :]
    a2 = bn2_g / jnp.sqrt(bn2_v + _EPS)
    W2s = W2 * a2[None, :]
    c2 = ((b2 - bn2_m) * a2 + bn2_b)[None, :]

    pooled = _make_pool(B, H, D)(idx, emb)

    BB = 512
    grid = (B // BB,)
    loc, scale = pl.pallas_call(
        _mlp_body,
        grid=grid,
        in_specs=[
            pl.BlockSpec((BB, H), lambda i: (i, 0)),
            pl.BlockSpec((BB, D), lambda i: (i, 0)),
            pl.BlockSpec((D, HID), lambda i: (0, 0)),
            pl.BlockSpec((1, HID), lambda i: (0, 0)),
            pl.BlockSpec((1, HID), lambda i: (0, 0)),
            pl.BlockSpec((HID, 2 * D), lambda i: (0, 0)),
            pl.BlockSpec((1, 2 * D), lambda i: (0, 0)),
        ],
        out_specs=[
            pl.BlockSpec((BB, D), lambda i: (i, 0)),
            pl.BlockSpec((BB, D), lambda i: (i, 0)),
        ],
        out_shape=[
            jax.ShapeDtypeStruct((B, D), jnp.float32),
            jax.ShapeDtypeStruct((B, D), jnp.float32),
        ],
    )(idx, pooled, W1s[:D], W1s[D:D + 1], c1, W2s, c2)
    return loc, scale


# speedup vs baseline: 1.2341x; 1.0160x over previous
import functools

import jax
import jax.numpy as jnp
from jax import lax
from jax.experimental import pallas as pl
from jax.experimental.pallas import tpu as pltpu
from jax.experimental.pallas import tpu_sc as plsc

_EPS = 1e-5

_NUM_SC = 2
_NUM_SUBCORES = 16
_LANES = 16
_NW = _NUM_SC * _NUM_SUBCORES
_NBUF = 4


@functools.lru_cache(maxsize=None)
def _make_pool(B, H, D):
    BPW = B // _NW
    assert B % _NW == 0 and BPW % _NBUF == 0 and D % _LANES == 0
    C0 = min(104, H)
    C1 = H - C0
    NCH = D // _LANES
    mesh = plsc.VectorSubcoreMesh(core_axis_name="c", subcore_axis_name="s")

    @functools.partial(
        pl.kernel,
        out_type=jax.ShapeDtypeStruct((B, D), jnp.float32),
        mesh=mesh,
        scratch_types=[
            pltpu.VMEM((BPW, H), jnp.int32),
            pltpu.VMEM((_NBUF, H, D), jnp.float32),
            pltpu.VMEM((BPW, D), jnp.float32),
            pltpu.SemaphoreType.DMA,
        ] + [pltpu.SemaphoreType.DMA] * _NBUF,
        compiler_params=pltpu.CompilerParams(use_tc_tiling_on_sc=False),
    )
    def pool(idx_hbm, emb_hbm, out_hbm, idx_v, rows_v, out_v, sem_i, *sems):
        wid = lax.axis_index("s") * _NUM_SC + lax.axis_index("c")
        base = wid * BPW
        pltpu.async_copy(idx_hbm.at[pl.ds(base, BPW)], idx_v, sem_i).wait()

        def fire(b, p):
            pltpu.make_async_copy(
                emb_hbm.at[idx_v.at[b, pl.ds(0, C0)]],
                rows_v.at[p, pl.ds(0, C0)], sems[p]).start()
            if C1:
                pltpu.make_async_copy(
                    emb_hbm.at[idx_v.at[b, pl.ds(C0, C1)]],
                    rows_v.at[p, pl.ds(C0, C1)], sems[p]).start()

        def wait(p):
            pltpu.make_async_copy(
                emb_hbm.at[idx_v.at[0, pl.ds(0, C0)]],
                rows_v.at[p, pl.ds(0, C0)], sems[p]).wait()
            if C1:
                pltpu.make_async_copy(
                    emb_hbm.at[idx_v.at[0, pl.ds(C0, C1)]],
                    rows_v.at[p, pl.ds(C0, C1)], sems[p]).wait()

        def reduce(b, p):
            def body(i, accs):
                return tuple(
                    a + rows_v[p, i, pl.ds(_LANES * c, _LANES)]
                    for c, a in enumerate(accs)
                )
            init = tuple(jnp.zeros((_LANES,), jnp.float32) for _ in range(NCH))
            accs = lax.fori_loop(0, H, body, init, unroll=8)
            for c in range(NCH):
                out_v[b, pl.ds(_LANES * c, _LANES)] = accs[c]

        for p in range(_NBUF):
            fire(p, p)

        @pl.loop(0, BPW, step=_NBUF)
        def _(b):
            for p in range(_NBUF):
                wait(p)
                reduce(b + p, p)

                @pl.when(b + p + _NBUF < BPW)
                def _():
                    fire(b + p + _NBUF, p)

        pltpu.sync_copy(out_v, out_hbm.at[pl.ds(base, BPW)])

    return pool


def _mlp_body(idx_ref, pooled_ref, w1_ref, w1r_ref, c1_ref, w2_ref, c2_ref,
              loc_ref, scale_ref):
    D = pooled_ref.shape[1]
    cnt = jnp.sum((idx_ref[...] > 0).astype(jnp.float32), axis=1, keepdims=True)
    x = pooled_ref[...] / cnt
    h = jnp.dot(x, w1_ref[...], preferred_element_type=jnp.float32)
    h = h + jnp.log(cnt) * w1r_ref[...] + c1_ref[...]
    h = jnp.maximum(h, 0.0)
    o = jnp.dot(h, w2_ref[...], preferred_element_type=jnp.float32) + c2_ref[...]
    loc_ref[...] = o[:, :D]
    scale_ref[...] = jax.nn.softplus(o[:, D:])


def kernel(idx, read_depth, emb, W1, b1, bn1_g, bn1_b, bn1_m, bn1_v,
           W2, b2, bn2_g, bn2_b, bn2_m, bn2_v):
    del read_depth
    idx = idx.astype(jnp.int32)
    B, H = idx.shape
    D = emb.shape[1]
    HID = W1.shape[1]

    a1 = bn1_g / jnp.sqrt(bn1_v + _EPS)
    W1s = W1 * a1[None, :]
    c1 = ((b1 - bn1_m) * a1 + bn1_b)[None, :]
    a2 = bn2_g / jnp.sqrt(bn2_v + _EPS)
    W2s = W2 * a2[None, :]
    c2 = ((b2 - bn2_m) * a2 + bn2_b)[None, :]

    pooled = _make_pool(B, H, D)(idx, emb)

    BB = 1024
    grid = (B // BB,)
    loc, scale = pl.pallas_call(
        _mlp_body,
        grid=grid,
        in_specs=[
            pl.BlockSpec((BB, H), lambda i: (i, 0)),
            pl.BlockSpec((BB, D), lambda i: (i, 0)),
            pl.BlockSpec((D, HID), lambda i: (0, 0)),
            pl.BlockSpec((1, HID), lambda i: (0, 0)),
            pl.BlockSpec((1, HID), lambda i: (0, 0)),
            pl.BlockSpec((HID, 2 * D), lambda i: (0, 0)),
            pl.BlockSpec((1, 2 * D), lambda i: (0, 0)),
        ],
        out_specs=[
            pl.BlockSpec((BB, D), lambda i: (i, 0)),
            pl.BlockSpec((BB, D), lambda i: (i, 0)),
        ],
        out_shape=[
            jax.ShapeDtypeStruct((B, D), jnp.float32),
            jax.ShapeDtypeStruct((B, D), jnp.float32),
        ],
    )(idx, pooled, W1s[:D], W1s[D:D + 1], c1, W2s, c2)
    return loc, scale


# speedup vs baseline: 1.2390x; 1.0040x over previous
import functools

import jax
import jax.numpy as jnp
from jax import lax
from jax.experimental import pallas as pl
from jax.experimental.pallas import tpu as pltpu
from jax.experimental.pallas import tpu_sc as plsc

_EPS = 1e-5

_NUM_SC = 2
_NUM_SUBCORES = 16
_LANES = 16
_NW = _NUM_SC * _NUM_SUBCORES
_NBUF = 4


@functools.lru_cache(maxsize=None)
def _make_pool(B, H, D):
    BPW = B // _NW
    assert B % _NW == 0 and BPW % _NBUF == 0 and D % _LANES == 0
    C0 = min(104, H)
    C1 = H - C0
    NCH = D // _LANES
    mesh = plsc.VectorSubcoreMesh(core_axis_name="c", subcore_axis_name="s")

    @functools.partial(
        pl.kernel,
        out_type=jax.ShapeDtypeStruct((B, D), jnp.float32),
        mesh=mesh,
        scratch_types=[
            pltpu.VMEM((BPW, H), jnp.int32),
            pltpu.VMEM((_NBUF, H, D), jnp.float32),
            pltpu.VMEM((BPW, D), jnp.float32),
            pltpu.SemaphoreType.DMA,
        ] + [pltpu.SemaphoreType.DMA] * _NBUF,
        compiler_params=pltpu.CompilerParams(use_tc_tiling_on_sc=False),
    )
    def pool(idx_hbm, emb_hbm, out_hbm, idx_v, rows_v, out_v, sem_i, *sems):
        wid = lax.axis_index("s") * _NUM_SC + lax.axis_index("c")
        base = wid * BPW
        pltpu.async_copy(idx_hbm.at[pl.ds(base, BPW)], idx_v, sem_i).wait()

        def fire(b, p):
            pltpu.make_async_copy(
                emb_hbm.at[idx_v.at[b, pl.ds(0, C0)]],
                rows_v.at[p, pl.ds(0, C0)], sems[p]).start()
            if C1:
                pltpu.make_async_copy(
                    emb_hbm.at[idx_v.at[b, pl.ds(C0, C1)]],
                    rows_v.at[p, pl.ds(C0, C1)], sems[p]).start()

        def wait(p):
            pltpu.make_async_copy(
                emb_hbm.at[idx_v.at[0, pl.ds(0, C0)]],
                rows_v.at[p, pl.ds(0, C0)], sems[p]).wait()
            if C1:
                pltpu.make_async_copy(
                    emb_hbm.at[idx_v.at[0, pl.ds(C0, C1)]],
                    rows_v.at[p, pl.ds(C0, C1)], sems[p]).wait()

        def reduce(b, p):
            def body(i, accs):
                return tuple(
                    a + rows_v[p, i, pl.ds(_LANES * c, _LANES)]
                    for c, a in enumerate(accs)
                )
            init = tuple(jnp.zeros((_LANES,), jnp.float32) for _ in range(NCH))
            accs = lax.fori_loop(0, H, body, init, unroll=8)
            for c in range(NCH):
                out_v[b, pl.ds(_LANES * c, _LANES)] = accs[c]

        for p in range(_NBUF):
            fire(p, p)

        @pl.loop(0, BPW, step=_NBUF)
        def _(b):
            for p in range(_NBUF):
                wait(p)
                reduce(b + p, p)

                @pl.when(b + p + _NBUF < BPW)
                def _():
                    fire(b + p + _NBUF, p)

        pltpu.sync_copy(out_v, out_hbm.at[pl.ds(base, BPW)])

    return pool


def _mlp_body(idx_ref, pooled_ref, w1_ref, w1r_ref, c1_ref, w2_ref, c2_ref,
              loc_ref, scale_ref):
    D = pooled_ref.shape[1]
    cnt = jnp.sum((idx_ref[...] > 0).astype(jnp.float32), axis=1, keepdims=True)
    x = pooled_ref[...] / cnt
    h = jnp.dot(x, w1_ref[...], preferred_element_type=jnp.float32)
    h = h + jnp.log(cnt) * w1r_ref[...] + c1_ref[...]
    h = jnp.maximum(h, 0.0)
    o = jnp.dot(h, w2_ref[...], preferred_element_type=jnp.float32) + c2_ref[...]
    loc_ref[...] = o[:, :D]
    scale_ref[...] = jax.nn.softplus(o[:, D:])


def kernel(idx, read_depth, emb, W1, b1, bn1_g, bn1_b, bn1_m, bn1_v,
           W2, b2, bn2_g, bn2_b, bn2_m, bn2_v):
    del read_depth
    idx = idx.astype(jnp.int32)
    B, H = idx.shape
    D = emb.shape[1]
    HID = W1.shape[1]

    a1 = bn1_g / jnp.sqrt(bn1_v + _EPS)
    W1s = W1 * a1[None, :]
    c1 = ((b1 - bn1_m) * a1 + bn1_b)[None, :]
    a2 = bn2_g / jnp.sqrt(bn2_v + _EPS)
    W2s = W2 * a2[None, :]
    c2 = ((b2 - bn2_m) * a2 + bn2_b)[None, :]

    pooled = _make_pool(B, H, D)(idx, emb)

    BB = 2048
    grid = (B // BB,)
    loc, scale = pl.pallas_call(
        _mlp_body,
        grid=grid,
        in_specs=[
            pl.BlockSpec((BB, H), lambda i: (i, 0)),
            pl.BlockSpec((BB, D), lambda i: (i, 0)),
            pl.BlockSpec((D, HID), lambda i: (0, 0)),
            pl.BlockSpec((1, HID), lambda i: (0, 0)),
            pl.BlockSpec((1, HID), lambda i: (0, 0)),
            pl.BlockSpec((HID, 2 * D), lambda i: (0, 0)),
            pl.BlockSpec((1, 2 * D), lambda i: (0, 0)),
        ],
        out_specs=[
            pl.BlockSpec((BB, D), lambda i: (i, 0)),
            pl.BlockSpec((BB, D), lambda i: (i, 0)),
        ],
        out_shape=[
            jax.ShapeDtypeStruct((B, D), jnp.float32),
            jax.ShapeDtypeStruct((B, D), jnp.float32),
        ],
    )(idx, pooled, W1s[:D], W1s[D:D + 1], c1, W2s, c2)
    return loc, scale
